# baseline (speedup 1.0000x reference)
import functools

import jax
import jax.numpy as jnp
from jax import lax
from jax.experimental import pallas as pl
from jax.experimental.pallas import tpu as pltpu
from jax.experimental.pallas import tpu_sc as plsc

_F = 26
_V = 100000
_D = 32
_B = 16384
_NW = 32
_BPW = _B // _NW
_CHUNK = 128
_NCH = _BPW // _CHUNK
_NBUF = 4


def _sc_encode(idx_all, tab_flat, nf):
    mesh = plsc.VectorSubcoreMesh(core_axis_name="c", subcore_axis_name="s")

    @functools.partial(
        pl.kernel,
        mesh=mesh,
        out_type=jax.ShapeDtypeStruct((_B, nf * _D), jnp.float32),
        scratch_types=[
            pltpu.VMEM((nf * _NCH, _CHUNK), jnp.int32),
            pltpu.VMEM((_NBUF, _BPW, _D), jnp.float32),
        ] + [pltpu.SemaphoreType.DMA] * (2 * _NBUF),
        compiler_params=pltpu.CompilerParams(use_tc_tiling_on_sc=False),
    )
    def k(idx_hbm, tab_hbm, out_hbm, idx_v, bufs, *sems):
        gsem = sems[:_NBUF]
        ssem = sems[_NBUF:]
        wid = lax.axis_index("s") * 2 + lax.axis_index("c")
        base = wid * _BPW

        pltpu.sync_copy(idx_hbm.at[wid], idx_v)

        def fire_gathers(f):
            b = f % _NBUF
            return [
                pltpu.async_copy(
                    tab_hbm.at[idx_v.at[f * _NCH + c]],
                    bufs.at[b, pl.ds(c * _CHUNK, _CHUNK)],
                    gsem[b],
                )
                for c in range(_NCH)
            ]

        gh = {}
        sh = {}
        for f in range(min(_NBUF, nf)):
            gh[f] = fire_gathers(f)
        for f in range(nf):
            b = f % _NBUF
            for h in gh.pop(f):
                h.wait()
            sh[f] = pltpu.async_copy(
                bufs.at[b], out_hbm.at[pl.ds(base, _BPW), pl.ds(f * _D, _D)],
                ssem[b],
            )
            nf2 = f + _NBUF
            if nf2 < nf:
                sh.pop(f).wait()
                gh[nf2] = fire_gathers(nf2)
        for f in sorted(sh):
            sh[f].wait()

    return k(idx_all, tab_flat)


_SPLITS = (7, 7, 6, 6)


def kernel(sparse_tensors, tables):
    idx = sparse_tensors.astype(jnp.int32)
    outs = []
    f0 = 0
    for nf in _SPLITS:
        idx_g = idx[f0:f0 + nf]
        offs = (jnp.arange(nf, dtype=jnp.int32) * _V)[:, None]
        gidx = (
            (idx_g + offs)
            .reshape(nf, _NW, _NCH, _CHUNK)
            .transpose(1, 0, 2, 3)
            .reshape(_NW, nf * _NCH, _CHUNK)
        )
        tab_flat = tables[f0:f0 + nf].reshape(nf * _V, _D)
        outs.append(_sc_encode(gidx, tab_flat, nf))
        f0 += nf
    return jnp.concatenate(outs, axis=1)
